# Initial kernel scaffold; baseline (speedup 1.0000x reference)
import jax
import jax.numpy as jnp
from jax.experimental import pallas as pl


def kernel(e, edge_index, prop_layers, alpha):
    raise NotImplementedError("write your pallas kernel here")


# speedup vs baseline: 408.2182x; 408.2182x over previous
import jax
import jax.numpy as jnp
from jax import lax
from jax.experimental import pallas as pl
from jax.experimental.pallas import tpu as pltpu
from jax.experimental.pallas import tpu_sc as plsc

NC = 2
NS = 16
NW = NC * NS


def _make_spmm(n_pad, n_edges, chunk, with_deg):
  span = n_pad // NS
  per_w = n_edges // NW
  n_chunks = per_w // chunk
  mesh = plsc.VectorSubcoreMesh(
      core_axis_name="c", subcore_axis_name="s",
      num_cores=NC, num_subcores=NS)

  out_type = [jax.ShapeDtypeStruct((NC * n_pad,), jnp.float32)]
  scratch = [
      pltpu.VMEM_SHARED((n_pad,), jnp.float32),
      pltpu.VMEM_SHARED((n_pad,), jnp.float32),
      pltpu.VMEM((chunk,), jnp.int32),
      pltpu.VMEM((chunk,), jnp.int32),
      pltpu.VMEM((chunk,), jnp.float32),
      pltpu.VMEM((n_pad // NS,), jnp.float32),
  ]
  if with_deg:
    out_type.append(jax.ShapeDtypeStruct((NC * n_pad,), jnp.float32))
    scratch += [
        pltpu.VMEM_SHARED((n_pad,), jnp.float32),
        pltpu.VMEM((chunk,), jnp.float32),
    ]

  def body(*refs):
    if with_deg:
      (ev_hbm, zeros_hbm, ones_hbm, row_hbm, col_hbm,
       acc_out, deg_out, ev_sh, acc_sh, ridx, cidx, vals, stage_v,
       deg_sh, ones_v) = refs
    else:
      (ev_hbm, zeros_hbm, row_hbm, col_hbm,
       acc_out, ev_sh, acc_sh, ridx, cidx, vals, stage_v) = refs

    c = lax.axis_index("c")
    s = lax.axis_index("s")
    off = s * span
    pltpu.sync_copy(ev_hbm.at[pl.ds(off, span)], stage_v)
    pltpu.sync_copy(stage_v, ev_sh.at[pl.ds(off, span)])
    pltpu.sync_copy(zeros_hbm.at[pl.ds(off, span)], stage_v)
    pltpu.sync_copy(stage_v, acc_sh.at[pl.ds(off, span)])
    if with_deg:
      pltpu.sync_copy(stage_v, deg_sh.at[pl.ds(off, span)])
      pltpu.sync_copy(ones_hbm, ones_v)
    plsc.subcore_barrier()

    base = (c * NS + s) * per_w
    for i in range(n_chunks):
      o = base + i * chunk
      pltpu.sync_copy(row_hbm.at[pl.ds(o, chunk)], ridx)
      pltpu.sync_copy(col_hbm.at[pl.ds(o, chunk)], cidx)
      pltpu.sync_copy(ev_sh.at[ridx], vals)
      pltpu.sync_copy(vals, acc_sh.at[cidx], add=True)
      if with_deg:
        pltpu.sync_copy(ones_v, deg_sh.at[cidx], add=True)
    plsc.subcore_barrier()

    oo = c * n_pad + off
    pltpu.sync_copy(acc_sh.at[pl.ds(off, span)], stage_v)
    pltpu.sync_copy(stage_v, acc_out.at[pl.ds(oo, span)])
    if with_deg:
      pltpu.sync_copy(deg_sh.at[pl.ds(off, span)], stage_v)
      pltpu.sync_copy(stage_v, deg_out.at[pl.ds(oo, span)])

  return pl.kernel(body, out_type=tuple(out_type), mesh=mesh,
                   scratch_types=tuple(scratch))


def kernel(e, edge_index, prop_layers=2, alpha=0.5):
  n = e.shape[0]
  n_edges = edge_index.shape[1]
  n_pad = -(-n // (NS * 8)) * (NS * 8)
  chunk = 10000
  assert n_edges % (NW * chunk) == 0

  row = edge_index[0].astype(jnp.int32)
  col = edge_index[1].astype(jnp.int32)
  e_pad = jnp.zeros((n_pad,), jnp.float32).at[:n].set(e.astype(jnp.float32))
  zeros = jnp.zeros((n_pad,), jnp.float32)
  ones = jnp.ones((chunk,), jnp.float32)

  spmm_deg = _make_spmm(n_pad, n_edges, chunk, with_deg=True)
  spmm = _make_spmm(n_pad, n_edges, chunk, with_deg=False)

  acc1_p, deg_p = spmm_deg(e_pad, zeros, ones, row, col)
  acc1 = acc1_p[:n_pad] + acc1_p[n_pad:]
  deg = deg_p[:n_pad] + deg_p[n_pad:]
  inv_deg = jnp.where(deg > 0, 1.0 / deg, 0.0)

  a = jnp.float32(alpha)
  ev1 = a * e_pad + (1.0 - a) * inv_deg * acc1
  acc2_p = spmm(ev1, zeros, row, col)
  if isinstance(acc2_p, (tuple, list)):
    acc2_p = acc2_p[0]
  acc2 = acc2_p[:n_pad] + acc2_p[n_pad:]
  ev2 = a * ev1 + (1.0 - a) * inv_deg * acc2
  return ev2[:n]
